# baseline (speedup 1.0000x reference)
import functools

import jax
import jax.numpy as jnp
from jax import lax
from jax.experimental import pallas as pl
from jax.experimental.pallas import tpu as pltpu
from jax.experimental.pallas import tpu_sc as plsc

B = 64
INPUT_SIZE = 2048
MEM_SIZE = 8192
CELL = 128
HEADS = 16
K = 8
PROJ = HEADS * CELL + CELL + 1
PROJ_PAD = 2304

NW = 32
B_PER_W = B // NW
TASKS = B_PER_W * HEADS
IDX_PER_W = TASKS * K
LANES = 16

NEG_INF = -3.4e38


def _proj_kernel(x_ref, w_ref, b_ref, y_ref):
    y_ref[...] = (
        jax.lax.dot_general(
            x_ref[...], w_ref[...], (((1,), (0,)), ((), ())),
            preferred_element_type=jnp.float32,
        )
        + b_ref[...]
    )


def _step_kernel(q_ref, mem_ref, wv_ref, gl_ref, w_ref, ti_ref, out_ref):
    q = q_ref[0]
    mem = mem_ref[0]

    scores = jax.lax.dot_general(
        q, mem, (((1,), (1,)), ((), ())), preferred_element_type=jnp.float32
    )

    iota = jax.lax.broadcasted_iota(jnp.int32, (HEADS, MEM_SIZE), 1)

    sc = scores
    topv = []
    topi = []
    chunk = MEM_SIZE // K
    for k in range(K):
        m = jnp.max(sc, axis=1, keepdims=True)
        idx = jnp.min(jnp.where(sc == m, iota, MEM_SIZE), axis=1, keepdims=True)
        topv.append(m)
        topi.append(idx)
        if k < K - 1:
            sc = jnp.where(iota == idx, NEG_INF, sc)
        out_ref[0, pl.ds(k * chunk, chunk), :] = mem_ref[0, pl.ds(k * chunk, chunk), :]

    v = jnp.concatenate(topv, axis=1)
    w = jax.nn.softmax(v, axis=1)
    w_ref[...] = jnp.broadcast_to(w[:, :, None], (HEADS, K, LANES))[None]

    base = pl.program_id(0) * MEM_SIZE
    ti = jnp.concatenate(topi, axis=1) + base
    ti_ref[...] = ti[None]

    pos = topi[0][0, 0]
    gate = jax.nn.sigmoid(gl_ref[0, 0, 0])
    gw = gate * wv_ref[0, 0]
    row = mem_ref[0, pl.ds(pos, 1), :] + gw[None, :]
    out_ref[0, pl.ds(pos, 1), :] = row


def _read_kernel(mem_hbm, ti_hbm, w_hbm, rv_hbm, idx_v, w_v, rows_v, out_v, sem):
    wid = lax.axis_index("s") * 2 + lax.axis_index("c")

    pltpu.sync_copy(ti_hbm.at[wid], idx_v)
    pltpu.sync_copy(w_hbm.at[wid], w_v)
    pltpu.async_copy(mem_hbm.at[idx_v], rows_v, sem).wait()

    def body(t, carry):
        acc = [jnp.zeros((LANES,), jnp.float32) for _ in range(CELL // LANES)]
        for k in range(K):
            wk = w_v[pl.ds((t * K + k) * LANES, LANES)]
            for c in range(CELL // LANES):
                rc = rows_v[t * K + k, pl.ds(c * LANES, LANES)]
                acc[c] = acc[c] + wk * rc
        for c in range(CELL // LANES):
            out_v[pl.ds(t * CELL + c * LANES, LANES)] = acc[c]
        return carry

    lax.fori_loop(0, TASKS, body, None)
    pltpu.sync_copy(out_v, rv_hbm.at[wid])


@functools.partial(jax.jit, static_argnames=("interpret",))
def kernel(x, memory, Wq, bq, Wv, bv, Wg, bg, interpret=False):
    w_all = jnp.zeros((INPUT_SIZE, PROJ_PAD), jnp.float32)
    w_all = w_all.at[:, : HEADS * CELL].set(Wq)
    w_all = w_all.at[:, HEADS * CELL : HEADS * CELL + CELL].set(Wv)
    w_all = w_all.at[:, HEADS * CELL + CELL : PROJ].set(Wg)
    b_all = jnp.zeros((1, PROJ_PAD), jnp.float32)
    b_all = b_all.at[0, : HEADS * CELL].set(bq)
    b_all = b_all.at[0, HEADS * CELL : HEADS * CELL + CELL].set(bv)
    b_all = b_all.at[0, HEADS * CELL + CELL : PROJ].set(bg)

    y = pl.pallas_call(
        _proj_kernel,
        out_shape=jax.ShapeDtypeStruct((B, PROJ_PAD), jnp.float32),
        interpret=interpret,
    )(x, w_all, b_all)

    queries = y[:, : HEADS * CELL].reshape(B, HEADS, CELL)
    wv = y[:, HEADS * CELL : HEADS * CELL + CELL].reshape(B, 1, CELL)
    gl = y[:, HEADS * CELL + CELL : PROJ].reshape(B, 1, 1)

    w8, ti8, new_mem = pl.pallas_call(
        _step_kernel,
        grid=(B,),
        in_specs=[
            pl.BlockSpec((1, HEADS, CELL), lambda b: (b, 0, 0)),
            pl.BlockSpec((1, MEM_SIZE, CELL), lambda b: (b, 0, 0)),
            pl.BlockSpec((1, 1, CELL), lambda b: (b, 0, 0)),
            pl.BlockSpec((1, 1, 1), lambda b: (b, 0, 0)),
        ],
        out_specs=[
            pl.BlockSpec((1, HEADS, K, LANES), lambda b: (b, 0, 0, 0)),
            pl.BlockSpec((1, HEADS, K), lambda b: (b, 0, 0)),
            pl.BlockSpec((1, MEM_SIZE, CELL), lambda b: (b, 0, 0)),
        ],
        out_shape=[
            jax.ShapeDtypeStruct((B, HEADS, K, LANES), jnp.float32),
            jax.ShapeDtypeStruct((B, HEADS, K), jnp.int32),
            jax.ShapeDtypeStruct((B, MEM_SIZE, CELL), jnp.float32),
        ],
        interpret=interpret,
    )(queries, memory, wv, gl)

    mem2d = memory.reshape(B * MEM_SIZE, CELL)
    ti32 = ti8.reshape(NW, IDX_PER_W)
    w32 = w8.reshape(NW, IDX_PER_W * LANES)

    read_k = functools.partial(
        pl.kernel,
        out_type=jax.ShapeDtypeStruct((NW, TASKS * CELL), jnp.float32),
        mesh=plsc.VectorSubcoreMesh(core_axis_name="c", subcore_axis_name="s"),
        scratch_types=[
            pltpu.VMEM((IDX_PER_W,), jnp.int32),
            pltpu.VMEM((IDX_PER_W * LANES,), jnp.float32),
            pltpu.VMEM((IDX_PER_W, CELL), jnp.float32),
            pltpu.VMEM((TASKS * CELL,), jnp.float32),
            pltpu.SemaphoreType.DMA,
        ],
        interpret=interpret,
    )(_read_kernel)

    rv32 = read_k(mem2d, ti32, w32)
    rv = rv32.reshape(B, HEADS, CELL)
    return rv, new_mem


# speedup vs baseline: 1.1978x; 1.1978x over previous
import functools

import jax
import jax.numpy as jnp
from jax import lax
from jax.experimental import pallas as pl
from jax.experimental.pallas import tpu as pltpu
from jax.experimental.pallas import tpu_sc as plsc

B = 64
INPUT_SIZE = 2048
MEM_SIZE = 8192
CELL = 128
HEADS = 16
K = 8
PROJ = HEADS * CELL + CELL + 1
PROJ_PAD = 2304

NW = 32
B_PER_W = B // NW
TASKS = B_PER_W * HEADS
IDX_PER_W = TASKS * K
LANES = 16

NEG_INF = -3.4e38
BBLK = 2


def _proj_kernel(x_ref, w_ref, b_ref, y_ref):
    y_ref[...] = (
        jax.lax.dot_general(
            x_ref[...], w_ref[...], (((1,), (0,)), ((), ())),
            preferred_element_type=jnp.float32,
        )
        + b_ref[...]
    )


def _step_kernel(q_ref, mem_ref, wv_ref, gl_ref, w_ref, ti_ref, out_ref):
    iota = jax.lax.broadcasted_iota(jnp.int32, (HEADS, MEM_SIZE), 1)
    chunk = MEM_SIZE // K
    for b2 in range(BBLK):
        q = q_ref[b2]
        mem = mem_ref[b2]

        scores = jax.lax.dot_general(
            q, mem, (((1,), (1,)), ((), ())),
            preferred_element_type=jnp.float32,
        )

        sc = scores
        topv = []
        topi = []
        for k in range(K):
            m = jnp.max(sc, axis=1, keepdims=True)
            idx = jnp.min(jnp.where(sc == m, iota, MEM_SIZE), axis=1,
                          keepdims=True)
            topv.append(m)
            topi.append(idx)
            if k < K - 1:
                sc = jnp.where(iota == idx, NEG_INF, sc)
            out_ref[b2, pl.ds(k * chunk, chunk), :] = (
                mem_ref[b2, pl.ds(k * chunk, chunk), :])

        v = jnp.concatenate(topv, axis=1)
        w = jax.nn.softmax(v, axis=1)
        w_ref[b2] = jnp.broadcast_to(w[:, :, None], (HEADS, K, LANES))

        base = (pl.program_id(0) * BBLK + b2) * MEM_SIZE
        ti_ref[b2] = jnp.concatenate(topi, axis=1) + base

        pos = topi[0][0, 0]
        gate = jax.nn.sigmoid(gl_ref[b2, 0, 0])
        gw = gate * wv_ref[b2, 0]
        row = mem_ref[b2, pl.ds(pos, 1), :] + gw[None, :]
        out_ref[b2, pl.ds(pos, 1), :] = row


def _read_kernel(mem_hbm, ti_hbm, w_hbm, rv_hbm, idx_v, w_v, rows_v, out_v, sem):
    wid = lax.axis_index("s") * 2 + lax.axis_index("c")

    pltpu.sync_copy(ti_hbm.at[wid], idx_v)
    pltpu.sync_copy(w_hbm.at[wid], w_v)
    pltpu.async_copy(mem_hbm.at[idx_v], rows_v, sem).wait()

    def body(t, carry):
        acc = [jnp.zeros((LANES,), jnp.float32) for _ in range(CELL // LANES)]
        for k in range(K):
            wk = w_v[pl.ds((t * K + k) * LANES, LANES)]
            for c in range(CELL // LANES):
                rc = rows_v[t * K + k, pl.ds(c * LANES, LANES)]
                acc[c] = acc[c] + wk * rc
        for c in range(CELL // LANES):
            out_v[pl.ds(t * CELL + c * LANES, LANES)] = acc[c]
        return carry

    lax.fori_loop(0, TASKS, body, None)
    pltpu.sync_copy(out_v, rv_hbm.at[wid])


@functools.partial(jax.jit, static_argnames=("interpret",))
def kernel(x, memory, Wq, bq, Wv, bv, Wg, bg, interpret=False):
    w_all = jnp.zeros((INPUT_SIZE, PROJ_PAD), jnp.float32)
    w_all = w_all.at[:, : HEADS * CELL].set(Wq)
    w_all = w_all.at[:, HEADS * CELL : HEADS * CELL + CELL].set(Wv)
    w_all = w_all.at[:, HEADS * CELL + CELL : PROJ].set(Wg)
    b_all = jnp.zeros((1, PROJ_PAD), jnp.float32)
    b_all = b_all.at[0, : HEADS * CELL].set(bq)
    b_all = b_all.at[0, HEADS * CELL : HEADS * CELL + CELL].set(bv)
    b_all = b_all.at[0, HEADS * CELL + CELL : PROJ].set(bg)

    y = pl.pallas_call(
        _proj_kernel,
        out_shape=jax.ShapeDtypeStruct((B, PROJ_PAD), jnp.float32),
        interpret=interpret,
    )(x, w_all, b_all)

    queries = y[:, : HEADS * CELL].reshape(B, HEADS, CELL)
    wv = y[:, HEADS * CELL : HEADS * CELL + CELL].reshape(B, 1, CELL)
    gl = y[:, HEADS * CELL + CELL : PROJ].reshape(B, 1, 1)

    w8, ti8, new_mem = pl.pallas_call(
        _step_kernel,
        grid=(B // BBLK,),
        in_specs=[
            pl.BlockSpec((BBLK, HEADS, CELL), lambda b: (b, 0, 0)),
            pl.BlockSpec((BBLK, MEM_SIZE, CELL), lambda b: (b, 0, 0)),
            pl.BlockSpec((BBLK, 1, CELL), lambda b: (b, 0, 0)),
            pl.BlockSpec((BBLK, 1, 1), lambda b: (b, 0, 0)),
        ],
        out_specs=[
            pl.BlockSpec((BBLK, HEADS, K, LANES), lambda b: (b, 0, 0, 0)),
            pl.BlockSpec((BBLK, HEADS, K), lambda b: (b, 0, 0)),
            pl.BlockSpec((BBLK, MEM_SIZE, CELL), lambda b: (b, 0, 0)),
        ],
        out_shape=[
            jax.ShapeDtypeStruct((B, HEADS, K, LANES), jnp.float32),
            jax.ShapeDtypeStruct((B, HEADS, K), jnp.int32),
            jax.ShapeDtypeStruct((B, MEM_SIZE, CELL), jnp.float32),
        ],
        interpret=interpret,
    )(queries, memory, wv, gl)

    mem2d = memory.reshape(B * MEM_SIZE, CELL)
    ti32 = ti8.reshape(NW, IDX_PER_W)
    w32 = w8.reshape(NW, IDX_PER_W * LANES)

    read_k = functools.partial(
        pl.kernel,
        out_type=jax.ShapeDtypeStruct((NW, TASKS * CELL), jnp.float32),
        mesh=plsc.VectorSubcoreMesh(core_axis_name="c", subcore_axis_name="s"),
        scratch_types=[
            pltpu.VMEM((IDX_PER_W,), jnp.int32),
            pltpu.VMEM((IDX_PER_W * LANES,), jnp.float32),
            pltpu.VMEM((IDX_PER_W, CELL), jnp.float32),
            pltpu.VMEM((TASKS * CELL,), jnp.float32),
            pltpu.SemaphoreType.DMA,
        ],
        interpret=interpret,
    )(_read_kernel)

    rv32 = read_k(mem2d, ti32, w32)
    rv = rv32.reshape(B, HEADS, CELL)
    return rv, new_mem
